# baseline (speedup 1.0000x reference)
import functools

import jax
import jax.numpy as jnp
from jax import lax
from jax.experimental import pallas as pl
from jax.experimental.pallas import tpu as pltpu
from jax.experimental.pallas import tpu_sc as plsc

DMODEL = 64
SCALE = 8.0
LANES = 16
NC, NS = 2, 16
NW = NC * NS
B = 4096 * 200
ROWS_PER_W = B // NW
CHUNK = 256
NCHUNK = ROWS_PER_W // CHUNK
T128 = 500000


@functools.partial(
    pl.kernel,
    out_type=jax.ShapeDtypeStruct((B, DMODEL), jnp.float32),
    mesh=plsc.VectorSubcoreMesh(core_axis_name="c", subcore_axis_name="s"),
    scratch_types=[
        pltpu.VMEM((CHUNK,), jnp.int32),
        pltpu.VMEM((CHUNK + LANES,), jnp.int32),
        pltpu.VMEM((CHUNK, 2 * DMODEL), jnp.float32),
        pltpu.VMEM((CHUNK, DMODEL), jnp.float32),
        pltpu.SemaphoreType.DMA,
    ],
)
def _pair_gather(idx_hbm, t128_hbm, out_hbm, idx_v, half_v, rows_v, sel_v, gsem):
    wid = lax.axis_index("s") * NC + lax.axis_index("c")
    base_w = wid * ROWS_PER_W
    t3 = t128_hbm

    def chunk_body(g, carry):
        base = base_w + g * CHUNK
        pltpu.sync_copy(idx_hbm.at[pl.ds(base, CHUNK)], idx_v)
        def prep(i, c2):
            sl = pl.ds(i * LANES, LANES)
            v = idx_v[sl]
            half_v[sl] = lax.rem(v, 2)
            idx_v[sl] = lax.div(v, 2)
            return c2

        lax.fori_loop(0, CHUNK // LANES, prep, 0)
        pltpu.async_copy(t128_hbm.at[idx_v], rows_v, gsem).wait()

        def sel(i, c2):
            h = half_v[pl.ds(i, LANES)][0]
            for j in range(DMODEL // LANES):
                sl = pl.ds(j * LANES, LANES)
                src = pl.ds(h * DMODEL + j * LANES, LANES)
                sel_v[i, sl] = rows_v[i, src] * SCALE
            return c2

        lax.fori_loop(0, CHUNK, sel, 0)
        pltpu.sync_copy(sel_v, out_hbm.at[pl.ds(base, CHUNK)])
        return carry

    lax.fori_loop(0, NCHUNK, chunk_body, 0)


def kernel(x, table):
    xf = x.reshape(-1)
    t128 = table.reshape(T128, 2 * DMODEL)
    out = _pair_gather(xf, t128)
    return out.reshape(x.shape + (DMODEL,))


# speedup vs baseline: 1.4833x; 1.4833x over previous
import functools

import jax
import jax.numpy as jnp
from jax import lax
from jax.experimental import pallas as pl
from jax.experimental.pallas import tpu as pltpu
from jax.experimental.pallas import tpu_sc as plsc

DMODEL = 64
SCALE = 8.0
LANES = 16
NC, NS = 2, 16
NW = NC * NS
DGRP = DMODEL // LANES

B = 4096 * 200
ROWS_PER_W = B // NW
CHUNK = 1024
NCHUNK = ROWS_PER_W // CHUNK


@functools.partial(
    pl.kernel,
    out_type=jax.ShapeDtypeStruct((B, DMODEL), jnp.float32),
    mesh=plsc.VectorSubcoreMesh(core_axis_name="c", subcore_axis_name="s"),
    compiler_params=pltpu.CompilerParams(use_tc_tiling_on_sc=False),
    scratch_types=[
        pltpu.VMEM((CHUNK,), jnp.int32),
        pltpu.VMEM((CHUNK, DMODEL), jnp.float32),
        pltpu.SemaphoreType.DMA,
    ],
)
def _emb_lookup(idx_hbm, table_hbm, out_hbm, idx_v, rows_v, gsem):
    wid = lax.axis_index("s") * NC + lax.axis_index("c")
    base_w = wid * ROWS_PER_W

    def chunk_body(g, carry):
        base = base_w + g * CHUNK
        pltpu.sync_copy(idx_hbm.at[pl.ds(base, CHUNK)], idx_v)
        pltpu.async_copy(table_hbm.at[idx_v], rows_v, gsem).wait()

        def row_body(r, c2):
            for dg in range(DGRP):
                sl = pl.ds(dg * LANES, LANES)
                rows_v[r, sl] = rows_v[r, sl] * SCALE
            return c2

        lax.fori_loop(0, CHUNK, row_body, 0)
        pltpu.sync_copy(rows_v, out_hbm.at[pl.ds(base, CHUNK)])
        return carry

    lax.fori_loop(0, NCHUNK, chunk_body, 0)


def kernel(x, table):
    xf = x.reshape(-1).astype(jnp.int32)
    out = _emb_lookup(xf, table)
    return out.reshape(x.shape + (DMODEL,))


# speedup vs baseline: 1.5690x; 1.0578x over previous
import functools

import jax
import jax.numpy as jnp
from jax import lax
from jax.experimental import pallas as pl
from jax.experimental.pallas import tpu as pltpu
from jax.experimental.pallas import tpu_sc as plsc

DMODEL = 64
SCALE = 8.0
LANES = 16
NC, NS = 2, 16
NW = NC * NS
DGRP = DMODEL // LANES

B = 4096 * 200
ROWS_PER_W = B // NW
CHUNK = 800
NCHUNK = ROWS_PER_W // CHUNK


@functools.partial(
    pl.kernel,
    out_type=jax.ShapeDtypeStruct((B, DMODEL), jnp.float32),
    mesh=plsc.VectorSubcoreMesh(core_axis_name="c", subcore_axis_name="s"),
    compiler_params=pltpu.CompilerParams(use_tc_tiling_on_sc=False),
    scratch_types=[
        pltpu.VMEM((CHUNK,), jnp.int32),
        pltpu.VMEM((CHUNK,), jnp.int32),
        pltpu.VMEM((CHUNK, DMODEL), jnp.float32),
        pltpu.VMEM((CHUNK, DMODEL), jnp.float32),
        pltpu.SemaphoreType.DMA,
        pltpu.SemaphoreType.DMA,
        pltpu.SemaphoreType.DMA,
        pltpu.SemaphoreType.DMA,
    ],
)
def _emb_lookup(idx_hbm, table_hbm, out_hbm, idx_a, idx_b, rows_a, rows_b,
                gsem_a, gsem_b, osem_a, osem_b):
    wid = lax.axis_index("s") * NC + lax.axis_index("c")
    base_w = wid * ROWS_PER_W
    bufs = ((idx_a, rows_a, gsem_a, osem_a), (idx_b, rows_b, gsem_b, osem_b))

    def start_gather(g):
        idx_v, rows_v, gsem, _ = bufs[g % 2]
        base = base_w + g * CHUNK
        pltpu.sync_copy(idx_hbm.at[pl.ds(base, CHUNK)], idx_v)
        return pltpu.async_copy(table_hbm.at[idx_v], rows_v, gsem)

    def scale(rows_v):
        def row_body(r, c2):
            for dg in range(DGRP):
                sl = pl.ds(dg * LANES, LANES)
                rows_v[r, sl] = rows_v[r, sl] * SCALE
            return c2

        lax.fori_loop(0, CHUNK, row_body, 0)

    gh = start_gather(0)
    oh = [None, None]
    for g in range(NCHUNK):
        idx_v, rows_v, gsem, osem = bufs[g % 2]
        nxt = None
        if g + 1 < NCHUNK:
            if oh[(g + 1) % 2] is not None:
                oh[(g + 1) % 2].wait()
                oh[(g + 1) % 2] = None
            nxt = start_gather(g + 1)
        gh.wait()
        scale(rows_v)
        base = base_w + g * CHUNK
        oh[g % 2] = pltpu.async_copy(
            rows_v, out_hbm.at[pl.ds(base, CHUNK)], osem
        )
        gh = nxt
    for h in oh:
        if h is not None:
            h.wait()


def kernel(x, table):
    xf = x.reshape(-1).astype(jnp.int32)
    out = _emb_lookup(xf, table)
    return out.reshape(x.shape + (DMODEL,))
